# baseline (speedup 1.0000x reference)
import jax
import jax.numpy as jnp
from jax import lax
from jax.experimental import pallas as pl
from jax.experimental.pallas import tpu as pltpu
from jax.experimental.pallas import tpu_sc as plsc

N = 10000
C = 128
OUTC = 64
W = 144
NPAD = 10048
NW = 32
B = 48
E_TOT = 320000 + N
E_PAD = 331776
EPW = E_PAD // NW
NB = EPW // B
NBH = NB // 2
KTOT = E_PAD // B
RPT = NPAD // 16


def _tables(h):
    norm = jnp.sqrt(jnp.sum(h * h, axis=1, keepdims=True))
    rinv = 1.0 / jnp.maximum(norm, 1e-12)
    t144 = jnp.concatenate([h, jnp.broadcast_to(rinv, (h.shape[0], 16))], axis=1)
    return t144, h * rinv


def _enc_body(x_ref, w1_ref, b1_ref, o144_ref, o128_ref):
    i = pl.program_id(0)
    h = lax.dot_general(x_ref[...], w1_ref[...], (((1,), (1,)), ((), ())),
                        preferred_element_type=jnp.float32)
    h = jnp.maximum(h + b1_ref[...], 0.0)
    rid = i * 256 + lax.broadcasted_iota(jnp.int32, (256, 1), 0)
    h = jnp.where(rid < N, h, 0.0)
    o144_ref[...], o128_ref[...] = _tables(h)


def _encode(x, W1, b1r):
    return pl.pallas_call(
        _enc_body,
        grid=(40,),
        in_specs=[
            pl.BlockSpec((256, C), lambda i: (i, 0)),
            pl.BlockSpec((C, C), lambda i: (0, 0)),
            pl.BlockSpec((1, C), lambda i: (0, 0)),
        ],
        out_specs=[
            pl.BlockSpec((256, W), lambda i: (i, 0)),
            pl.BlockSpec((256, C), lambda i: (i, 0)),
        ],
        out_shape=[
            jax.ShapeDtypeStruct((NPAD, W), jnp.float32),
            jax.ShapeDtypeStruct((NPAD, C), jnp.float32),
        ],
    )(x, W1, b1r)


def _comb_body(p0_ref, p1_ref, o144_ref, o128_ref):
    y = p0_ref[...] + p1_ref[...]
    h = y[:, :C] / (y[:, C:C + 1] + 1e-16)
    o144_ref[...], o128_ref[...] = _tables(h)


def _combine(p0, p1):
    return pl.pallas_call(
        _comb_body,
        grid=(40,),
        in_specs=[
            pl.BlockSpec((256, W), lambda i: (i, 0)),
            pl.BlockSpec((256, W), lambda i: (i, 0)),
        ],
        out_specs=[
            pl.BlockSpec((256, W), lambda i: (i, 0)),
            pl.BlockSpec((256, C), lambda i: (i, 0)),
        ],
        out_shape=[
            jax.ShapeDtypeStruct((NPAD, W), jnp.float32),
            jax.ShapeDtypeStruct((NPAD, C), jnp.float32),
        ],
    )(p0, p1)


def _head_body(p0_ref, p1_ref, w2_ref, b2_ref, o_ref):
    y = p0_ref[...] + p1_ref[...]
    h = y[:, :C] / (y[:, C:C + 1] + 1e-16)
    z = lax.dot_general(h, w2_ref[...], (((1,), (1,)), ((), ())),
                        preferred_element_type=jnp.float32)
    z = z + b2_ref[...]
    m = jnp.max(z, axis=1, keepdims=True)
    zz = z - m
    lse = jnp.log(jnp.sum(jnp.exp(zz), axis=1, keepdims=True))
    o_ref[...] = zz - lse


def _head(p0, p1, W2, b2r):
    return pl.pallas_call(
        _head_body,
        grid=(25,),
        in_specs=[
            pl.BlockSpec((400, W), lambda i: (i, 0)),
            pl.BlockSpec((400, W), lambda i: (i, 0)),
            pl.BlockSpec((OUTC, C), lambda i: (0, 0)),
            pl.BlockSpec((1, OUTC), lambda i: (0, 0)),
        ],
        out_specs=pl.BlockSpec((400, OUTC), lambda i: (i, 0)),
        out_shape=jax.ShapeDtypeStruct((N, OUTC), jnp.float32),
    )(p0, p1, W2, b2r)


def _prop_body(t144_hbm, t128_hbm, idx_hbm, beta_hbm, out_hbm,
               idxA, idxB, didxA, didxB, srowsA, srowsB, drowsA, drowsB,
               orowsA, orowsB, betav,
               acc, semSA, semSB, semDA, semDB, semOA, semOB, semIA, semIB):
    cid = lax.axis_index("c")
    sid = lax.axis_index("s")
    wid = sid * 2 + cid

    zero16 = jnp.zeros((16,), jnp.float32)

    def zrow(r, carry):
        for k in range(W // 16):
            srowsA[r, pl.ds(16 * k, 16)] = zero16
        return carry

    lax.fori_loop(0, B, zrow, None)

    r0 = sid * RPT
    for t in range(RPT // B):
        pltpu.sync_copy(srowsA, acc.at[pl.ds(r0 + t * B, B)])
    pltpu.sync_copy(srowsA.at[pl.ds(0, RPT % B)],
                    acc.at[pl.ds(r0 + (RPT // B) * B, RPT % B)])

    pltpu.sync_copy(beta_hbm, betav)
    plsc.subcore_barrier()

    bv = betav[...]
    babs = jnp.abs(bv)
    lanes = lax.iota(jnp.int32, 16)
    m0 = lanes == 0
    perms = [lanes ^ s for s in (8, 4, 2, 1)]

    def hsum(v):
        for p in perms:
            v = v + v.at[p].get(mode="promise_in_bounds")
        return v

    def compute(srows, drows, orows):
        @plsc.parallel_loop(0, B, unroll=4)
        def edge(j):
            s = [srows[j, pl.ds(16 * k, 16)] for k in range(8)]
            d = [drows[j, pl.ds(16 * k, 16)] for k in range(8)]
            p01 = s[0] * d[0] + s[1] * d[1]
            p23 = s[2] * d[2] + s[3] * d[3]
            p45 = s[4] * d[4] + s[5] * d[5]
            p67 = s[6] * d[6] + s[7] * d[7]
            a = hsum((p01 + p23) + (p45 + p67)) * srows[j, pl.ds(C, 16)]
            ev = jnp.exp(bv * a - babs)
            for k in range(8):
                orows[j, pl.ds(16 * k, 16)] = s[k] * ev
            orows[j, pl.ds(C, 16)] = jnp.where(m0, ev, 0.0)

    def start_gathers(idx, srows, drows, semS, semD):
        pltpu.async_copy(t144_hbm.at[idx.at[0]], srows, semS)
        pltpu.async_copy(t128_hbm.at[idx.at[1]], drows, semD)

    kbase = wid * NB
    pltpu.sync_copy(idx_hbm.at[kbase], idxA)
    pltpu.async_copy(idx_hbm.at[kbase + 1], idxB, semIB)
    start_gathers(idxA, srowsA, drowsA, semSA, semDA)

    def stage(t, k_next, idx, didx, srows, drows, orows,
              semS, semD, semO, semI):
        pltpu.make_async_copy(t144_hbm.at[idx.at[0]], srows, semS).wait()
        pltpu.make_async_copy(t128_hbm.at[idx.at[1]], drows, semD).wait()

        @pl.when(t > 0)
        def _():
            pltpu.make_async_copy(orows, acc.at[didx], semO).wait()

        for k in range(B // 16):
            didx[pl.ds(16 * k, 16)] = idx[1, pl.ds(16 * k, 16)]
        pltpu.async_copy(idx_hbm.at[k_next], idx, semI)
        compute(srows, drows, orows)
        pltpu.async_copy(orows, acc.at[didx], semO, add=True)

    def pair_body(t, carry):
        k0 = kbase + 2 * t
        pltpu.make_async_copy(idx_hbm.at[k0 + 1], idxB, semIB).wait()
        start_gathers(idxB, srowsB, drowsB, semSB, semDB)
        stage(t, k0 + 2, idxA, didxA, srowsA, drowsA, orowsA,
              semSA, semDA, semOA, semIA)
        pltpu.make_async_copy(idx_hbm.at[k0 + 2], idxA, semIA).wait()
        start_gathers(idxA, srowsA, drowsA, semSA, semDA)
        stage(t, k0 + 3, idxB, didxB, srowsB, drowsB, orowsB,
              semSB, semDB, semOB, semIB)
        return carry

    lax.fori_loop(0, NBH, pair_body, None)

    pltpu.make_async_copy(orowsA, acc.at[didxA], semOA).wait()
    pltpu.make_async_copy(orowsB, acc.at[didxB], semOB).wait()
    pltpu.make_async_copy(t144_hbm.at[idxA.at[0]], srowsA, semSA).wait()
    pltpu.make_async_copy(t128_hbm.at[idxA.at[1]], drowsA, semDA).wait()
    pltpu.make_async_copy(idx_hbm.at[kbase], idxB, semIB).wait()

    plsc.subcore_barrier()
    pltpu.sync_copy(acc.at[pl.ds(r0, RPT)], out_hbm.at[cid, pl.ds(r0, RPT)])


def _prop():
  return pl.kernel(
    _prop_body,
    out_type=jax.ShapeDtypeStruct((2, NPAD, W), jnp.float32),
    mesh=plsc.VectorSubcoreMesh(core_axis_name="c", subcore_axis_name="s"),
    compiler_params=pltpu.CompilerParams(use_tc_tiling_on_sc=False),
    scratch_types=[
        pltpu.VMEM((2, B), jnp.int32),
        pltpu.VMEM((2, B), jnp.int32),
        pltpu.VMEM((B,), jnp.int32),
        pltpu.VMEM((B,), jnp.int32),
        pltpu.VMEM((B, W), jnp.float32),
        pltpu.VMEM((B, W), jnp.float32),
        pltpu.VMEM((B, C), jnp.float32),
        pltpu.VMEM((B, C), jnp.float32),
        pltpu.VMEM((B, W), jnp.float32),
        pltpu.VMEM((B, W), jnp.float32),
        pltpu.VMEM((16,), jnp.float32),
        pltpu.VMEM_SHARED((NPAD, W), jnp.float32),
        pltpu.SemaphoreType.DMA,
        pltpu.SemaphoreType.DMA,
        pltpu.SemaphoreType.DMA,
        pltpu.SemaphoreType.DMA,
        pltpu.SemaphoreType.DMA,
        pltpu.SemaphoreType.DMA,
        pltpu.SemaphoreType.DMA,
        pltpu.SemaphoreType.DMA,
    ],
  )


@jax.jit
def kernel(x, edge_index, W1, b1, W2, b2, beta2):
    loop = jnp.arange(N, dtype=jnp.int32)
    padv = jnp.zeros((E_PAD - E_TOT,), dtype=jnp.int32) + N
    src = jnp.concatenate([edge_index[0].astype(jnp.int32), loop, padv])
    dst = jnp.concatenate([edge_index[1].astype(jnp.int32), loop, padv])
    idx = jnp.stack([src.reshape(KTOT, B), dst.reshape(KTOT, B)], axis=1)
    idx = jnp.concatenate([idx, jnp.zeros((2, 2, B), jnp.int32)], axis=0)
    b1r = b1.reshape(1, C)
    b2r = b2.reshape(1, OUTC)
    beta1v = jnp.full((16,), 1.0, dtype=jnp.float32)
    beta2v = jnp.broadcast_to(beta2.astype(jnp.float32), (16,))

    t144_0, t128_0 = _encode(x, W1, b1r)
    p = _prop()(t144_0, t128_0, idx, beta1v)
    t144_1, t128_1 = _combine(p[0], p[1])
    q = _prop()(t144_1, t128_1, idx, beta2v)
    return _head(q[0], q[1], W2, b2r)


# speedup vs baseline: 1.2933x; 1.2933x over previous
import jax
import jax.numpy as jnp
from jax import lax
from jax.experimental import pallas as pl
from jax.experimental.pallas import tpu as pltpu
from jax.experimental.pallas import tpu_sc as plsc

N = 10000
C = 128
OUTC = 64
W = 144
WS = 160
NPAD = 10048
NW = 32
B = 64
E_TOT = 320000 + N
E_PAD = 331776
EPW = E_PAD // NW
NB = EPW // B
NBH = NB // 2
KTOT = E_PAD // B
RPT = NPAD // 16


def _tables(h):
    norm = jnp.sqrt(jnp.sum(h * h, axis=1, keepdims=True))
    rinv = 1.0 / jnp.maximum(norm, 1e-12)
    t160 = jnp.concatenate(
        [h, jnp.broadcast_to(rinv, (h.shape[0], 32))], axis=1).astype(jnp.bfloat16)
    return t160, (h * rinv).astype(jnp.bfloat16)


def _enc_body(x_ref, w1_ref, b1_ref, o144_ref, o128_ref):
    i = pl.program_id(0)
    h = lax.dot_general(x_ref[...], w1_ref[...], (((1,), (1,)), ((), ())),
                        preferred_element_type=jnp.float32)
    h = jnp.maximum(h + b1_ref[...], 0.0)
    rid = i * 256 + lax.broadcasted_iota(jnp.int32, (256, 1), 0)
    h = jnp.where(rid < N, h, 0.0)
    o144_ref[...], o128_ref[...] = _tables(h)


def _encode(x, W1, b1r):
    return pl.pallas_call(
        _enc_body,
        grid=(40,),
        in_specs=[
            pl.BlockSpec((256, C), lambda i: (i, 0)),
            pl.BlockSpec((C, C), lambda i: (0, 0)),
            pl.BlockSpec((1, C), lambda i: (0, 0)),
        ],
        out_specs=[
            pl.BlockSpec((256, WS), lambda i: (i, 0)),
            pl.BlockSpec((256, C), lambda i: (i, 0)),
        ],
        out_shape=[
            jax.ShapeDtypeStruct((NPAD, WS), jnp.bfloat16),
            jax.ShapeDtypeStruct((NPAD, C), jnp.bfloat16),
        ],
    )(x, W1, b1r)


def _comb_body(p0_ref, p1_ref, o144_ref, o128_ref):
    y = p0_ref[...] + p1_ref[...]
    h = y[:, :C] / (y[:, C:C + 1] + 1e-16)
    o144_ref[...], o128_ref[...] = _tables(h)


def _combine(p0, p1):
    return pl.pallas_call(
        _comb_body,
        grid=(40,),
        in_specs=[
            pl.BlockSpec((256, W), lambda i: (i, 0)),
            pl.BlockSpec((256, W), lambda i: (i, 0)),
        ],
        out_specs=[
            pl.BlockSpec((256, WS), lambda i: (i, 0)),
            pl.BlockSpec((256, C), lambda i: (i, 0)),
        ],
        out_shape=[
            jax.ShapeDtypeStruct((NPAD, WS), jnp.bfloat16),
            jax.ShapeDtypeStruct((NPAD, C), jnp.bfloat16),
        ],
    )(p0, p1)


def _head_body(p0_ref, p1_ref, w2_ref, b2_ref, o_ref):
    y = p0_ref[...] + p1_ref[...]
    h = y[:, :C] / (y[:, C:C + 1] + 1e-16)
    z = lax.dot_general(h, w2_ref[...], (((1,), (1,)), ((), ())),
                        preferred_element_type=jnp.float32)
    z = z + b2_ref[...]
    m = jnp.max(z, axis=1, keepdims=True)
    zz = z - m
    lse = jnp.log(jnp.sum(jnp.exp(zz), axis=1, keepdims=True))
    o_ref[...] = zz - lse


def _head(p0, p1, W2, b2r):
    return pl.pallas_call(
        _head_body,
        grid=(25,),
        in_specs=[
            pl.BlockSpec((400, W), lambda i: (i, 0)),
            pl.BlockSpec((400, W), lambda i: (i, 0)),
            pl.BlockSpec((OUTC, C), lambda i: (0, 0)),
            pl.BlockSpec((1, OUTC), lambda i: (0, 0)),
        ],
        out_specs=pl.BlockSpec((400, OUTC), lambda i: (i, 0)),
        out_shape=jax.ShapeDtypeStruct((N, OUTC), jnp.float32),
    )(p0, p1, W2, b2r)


def _prop_body(t144_hbm, t128_hbm, idx_hbm, beta_hbm, out_hbm,
               idxA, idxB, didxA, didxB, srowsA, srowsB, drowsA, drowsB,
               orowsA, orowsB, betav,
               acc, semSA, semSB, semDA, semDB, semOA, semOB, semIA, semIB):
    cid = lax.axis_index("c")
    sid = lax.axis_index("s")
    wid = sid * 2 + cid

    zero16 = jnp.zeros((16,), jnp.float32)

    def zrow(r, carry):
        for k in range(W // 16):
            orowsA[r, pl.ds(16 * k, 16)] = zero16
        return carry

    lax.fori_loop(0, B, zrow, None)

    r0 = sid * RPT
    for t in range(RPT // B):
        pltpu.sync_copy(orowsA, acc.at[pl.ds(r0 + t * B, B)])
    pltpu.sync_copy(orowsA.at[pl.ds(0, RPT % B)],
                    acc.at[pl.ds(r0 + (RPT // B) * B, RPT % B)])

    pltpu.sync_copy(beta_hbm, betav)
    plsc.subcore_barrier()

    bv = betav[...]
    babs = jnp.abs(bv)
    lanes = lax.iota(jnp.int32, 16)
    m0 = lanes == 0
    perms = [lanes ^ s for s in (8, 4, 2, 1)]

    def hsum(v):
        for p in perms:
            v = v + v.at[p].get(mode="promise_in_bounds")
        return v

    def compute(srows, drows, orows):
        @plsc.parallel_loop(0, B, unroll=4)
        def edge(j):
            s = [srows[j, pl.ds(32 * k, 32)] for k in range(4)]
            d = [drows[j, pl.ds(32 * k, 32)] for k in range(4)]
            c32 = (s[0] * d[0] + s[1] * d[1]) + (s[2] * d[2] + s[3] * d[3])
            ca, cb = plsc.unpack(c32, format=plsc.PackFormat.INTERLEAVED)
            ra, rb = plsc.unpack(srows[j, pl.ds(C, 32)],
                                 format=plsc.PackFormat.INTERLEAVED)
            a = hsum(ca + cb) * ra
            ev = jnp.exp(bv * a - babs)
            for k in range(4):
                ua, ub = plsc.unpack(s[k], format=plsc.PackFormat.INTERLEAVED)
                orows[j, pl.ds(32 * k, 16)] = ua * ev
                orows[j, pl.ds(32 * k + 16, 16)] = ub * ev
            orows[j, pl.ds(C, 16)] = jnp.where(m0, ev, 0.0)

    def start_gathers(idx, srows, drows, semS, semD):
        pltpu.async_copy(t144_hbm.at[idx.at[0]], srows, semS)
        pltpu.async_copy(t128_hbm.at[idx.at[1]], drows, semD)

    kbase = wid * NB
    pltpu.sync_copy(idx_hbm.at[kbase], idxA)
    pltpu.async_copy(idx_hbm.at[kbase + 1], idxB, semIB)
    start_gathers(idxA, srowsA, drowsA, semSA, semDA)

    def stage(t, k_next, idx, didx, srows, drows, orows,
              semS, semD, semO, semI):
        pltpu.make_async_copy(t144_hbm.at[idx.at[0]], srows, semS).wait()
        pltpu.make_async_copy(t128_hbm.at[idx.at[1]], drows, semD).wait()

        @pl.when(t > 0)
        def _():
            pltpu.make_async_copy(orows, acc.at[didx], semO).wait()

        for k in range(B // 16):
            didx[pl.ds(16 * k, 16)] = idx[1, pl.ds(16 * k, 16)]
        pltpu.async_copy(idx_hbm.at[k_next], idx, semI)
        compute(srows, drows, orows)
        pltpu.async_copy(orows, acc.at[didx], semO, add=True)

    def pair_body(t, carry):
        k0 = kbase + 2 * t
        pltpu.make_async_copy(idx_hbm.at[k0 + 1], idxB, semIB).wait()
        start_gathers(idxB, srowsB, drowsB, semSB, semDB)
        stage(t, k0 + 2, idxA, didxA, srowsA, drowsA, orowsA,
              semSA, semDA, semOA, semIA)
        pltpu.make_async_copy(idx_hbm.at[k0 + 2], idxA, semIA).wait()
        start_gathers(idxA, srowsA, drowsA, semSA, semDA)
        stage(t, k0 + 3, idxB, didxB, srowsB, drowsB, orowsB,
              semSB, semDB, semOB, semIB)
        return carry

    lax.fori_loop(0, NBH, pair_body, None)

    pltpu.make_async_copy(orowsA, acc.at[didxA], semOA).wait()
    pltpu.make_async_copy(orowsB, acc.at[didxB], semOB).wait()
    pltpu.make_async_copy(t144_hbm.at[idxA.at[0]], srowsA, semSA).wait()
    pltpu.make_async_copy(t128_hbm.at[idxA.at[1]], drowsA, semDA).wait()
    pltpu.make_async_copy(idx_hbm.at[kbase], idxB, semIB).wait()

    plsc.subcore_barrier()
    pltpu.sync_copy(acc.at[pl.ds(r0, RPT)], out_hbm.at[cid, pl.ds(r0, RPT)])


def _prop():
  return pl.kernel(
    _prop_body,
    out_type=jax.ShapeDtypeStruct((2, NPAD, W), jnp.float32),
    mesh=plsc.VectorSubcoreMesh(core_axis_name="c", subcore_axis_name="s"),
    compiler_params=pltpu.CompilerParams(use_tc_tiling_on_sc=False, needs_layout_passes=False),
    scratch_types=[
        pltpu.VMEM((2, B), jnp.int32),
        pltpu.VMEM((2, B), jnp.int32),
        pltpu.VMEM((B,), jnp.int32),
        pltpu.VMEM((B,), jnp.int32),
        pltpu.VMEM((B, WS), jnp.bfloat16),
        pltpu.VMEM((B, WS), jnp.bfloat16),
        pltpu.VMEM((B, C), jnp.bfloat16),
        pltpu.VMEM((B, C), jnp.bfloat16),
        pltpu.VMEM((B, W), jnp.float32),
        pltpu.VMEM((B, W), jnp.float32),
        pltpu.VMEM((16,), jnp.float32),
        pltpu.VMEM_SHARED((NPAD, W), jnp.float32),
        pltpu.SemaphoreType.DMA,
        pltpu.SemaphoreType.DMA,
        pltpu.SemaphoreType.DMA,
        pltpu.SemaphoreType.DMA,
        pltpu.SemaphoreType.DMA,
        pltpu.SemaphoreType.DMA,
        pltpu.SemaphoreType.DMA,
        pltpu.SemaphoreType.DMA,
    ],
  )


import numpy as _np
_P = _np.empty((C,), dtype=_np.int32)
for _k in range(4):
    for _i in range(16):
        _P[32 * _k + _i] = 32 * _k + 2 * _i
        _P[32 * _k + 16 + _i] = 32 * _k + 2 * _i + 1
_PP = _P[_P]


@jax.jit
def kernel(x, edge_index, W1, b1, W2, b2, beta2):
    loop = jnp.arange(N, dtype=jnp.int32)
    padv = jnp.zeros((E_PAD - E_TOT,), dtype=jnp.int32) + N
    src = jnp.concatenate([edge_index[0].astype(jnp.int32), loop, padv])
    dst = jnp.concatenate([edge_index[1].astype(jnp.int32), loop, padv])
    idx = jnp.stack([src.reshape(KTOT, B), dst.reshape(KTOT, B)], axis=1)
    idx = jnp.concatenate([idx, jnp.zeros((2, 2, B), jnp.int32)], axis=0)
    b1r = b1.reshape(1, C)
    b2r = b2.reshape(1, OUTC)
    beta1v = jnp.full((16,), 1.0, dtype=jnp.float32)
    beta2v = jnp.broadcast_to(beta2.astype(jnp.float32), (16,))

    t144_0, t128_0 = _encode(x, W1, b1r)
    p = _prop()(t144_0, t128_0, idx, beta1v)
    t144_1, t128_1 = _combine(p[0], p[1])
    q = _prop()(t144_1, t128_1, idx, beta2v)
    return _head(q[0], q[1], W2[:, _PP], b2r)
